# baseline (speedup 1.0000x reference)
import functools

import jax
import jax.numpy as jnp
from jax import lax
from jax.experimental import pallas as pl
from jax.experimental.pallas import tpu as pltpu
from jax.experimental.pallas import tpu_sc as plsc

_EPS = 1e-12

_H = 1024
_B = 64
_S = 128
_NB = 196
_VFEAT = 2048

_NTOK = _B * _S
_NW = 32
_SPW = _S // _NW
_NL = _H // 16


def _rsqrt_nr(v):
    i = lax.bitcast_convert_type(v, jnp.int32)
    y = lax.bitcast_convert_type(jnp.int32(0x5F3759DF) - (i >> 1), jnp.float32)
    half = v * 0.5
    for _ in range(3):
        y = y * (1.5 - half * y * y)
    return y


def _text_sc(word_table, tok_t, fused_base, spos, gamma, beta):
    mesh = plsc.VectorSubcoreMesh(core_axis_name="c", subcore_axis_name="s")

    @functools.partial(
        pl.kernel,
        mesh=mesh,
        out_type=jax.ShapeDtypeStruct((_NTOK, _H), jnp.float32),
        scratch_types=[
            pltpu.VMEM((_SPW, _B), jnp.int32),
            pltpu.VMEM((_SPW, _B), jnp.int32),
            pltpu.VMEM((_B, _H), jnp.float32),
            pltpu.VMEM((_SPW, _H), jnp.float32),
            pltpu.VMEM((_H,), jnp.float32),
            pltpu.VMEM((_H,), jnp.float32),
            pltpu.SemaphoreType.DMA,
        ],
    )
    def k(table_hbm, tok_hbm, fb_hbm, spos_hbm, g_hbm, bt_hbm, out_hbm,
          idx_v, spos_v, rows_v, base_v, gam_v, bet_v, sem):
        wid = lax.axis_index("s") * 2 + lax.axis_index("c")
        s0 = wid * _SPW
        pltpu.sync_copy(tok_hbm.at[pl.ds(s0, _SPW)], idx_v)
        pltpu.sync_copy(spos_hbm.at[pl.ds(s0, _SPW)], spos_v)
        pltpu.sync_copy(fb_hbm.at[pl.ds(s0, _SPW)], base_v)
        pltpu.sync_copy(g_hbm, gam_v)
        pltpu.sync_copy(bt_hbm, bet_v)

        inv_h = jnp.full((16,), 1.0 / _H, jnp.float32)
        eps_v = jnp.full((16,), _EPS, jnp.float32)
        _dnums = lax.GatherDimensionNumbers(
            offset_dims=(), collapsed_slice_dims=(0,), start_index_map=(0,))
        _lane = lax.iota(jnp.int32, 16)
        _xor_idx = [(_lane ^ k).reshape(16, 1) for k in (8, 4, 2, 1)]

        def _splat_sum(v):
            for xi in _xor_idx:
                v = v + lax.gather(
                    v, xi, _dnums, (1,),
                    mode=lax.GatherScatterMode.PROMISE_IN_BOUNDS)
            return v

        for c in range(_SPW):
            pltpu.async_copy(table_hbm.at[idx_v.at[c]], rows_v, sem).wait()

            def row_body(r, _, c=c):
                acc = jnp.zeros((16,), jnp.float32)
                acc2 = jnp.zeros((16,), jnp.float32)
                for q in range(_NL):
                    sl = pl.ds(q * 16, 16)
                    x = rows_v[r, sl] + base_v[c, sl]
                    rows_v[r, sl] = x
                    acc = acc + x
                    acc2 = acc2 + x * x
                mu = _splat_sum(acc) * inv_h
                mu2 = _splat_sum(acc2) * inv_h
                rinv = _rsqrt_nr(mu2 - mu * mu + eps_v)
                for q in range(_NL):
                    sl = pl.ds(q * 16, 16)
                    rows_v[r, sl] = ((rows_v[r, sl] - mu) * rinv * gam_v[sl]
                                     + bet_v[sl])
                return _

            lax.fori_loop(0, _B, row_body, None)
            pltpu.async_copy(rows_v, out_hbm.at[spos_v.at[c]], sem).wait()

    return k(word_table, tok_t, fused_base, spos, gamma, beta)


def _visual_tc(x3, w, bias_row, gamma_row, beta_row):
    k, n = _VFEAT, _H
    bb = 4
    grid = (_B // bb,)

    def body(x_ref, w_ref, b_ref, g_ref, bt_ref, o_ref):
        for j in range(bb):
            acc = jnp.dot(x_ref[j], w_ref[...],
                          preferred_element_type=jnp.float32)
            y = acc + b_ref[...]
            mu = jnp.mean(y, axis=-1, keepdims=True)
            var = jnp.mean((y - mu) ** 2, axis=-1, keepdims=True)
            o_ref[j] = (y - mu) * lax.rsqrt(var + _EPS) * g_ref[...] + bt_ref[...]

    return pl.pallas_call(
        body,
        grid=grid,
        in_specs=[
            pl.BlockSpec((bb, _NB, k), lambda i: (i, 0, 0)),
            pl.BlockSpec((k, n), lambda i: (0, 0)),
            pl.BlockSpec((1, n), lambda i: (0, 0)),
            pl.BlockSpec((1, n), lambda i: (0, 0)),
            pl.BlockSpec((1, n), lambda i: (0, 0)),
        ],
        out_specs=pl.BlockSpec((bb, _NB, n), lambda i: (i, 0, 0)),
        out_shape=jax.ShapeDtypeStruct((_B, _NB, n), jnp.float32),
    )(x3, w, bias_row, gamma_row, beta_row)


def kernel(token_ids, image_feat, image_loc, word_table, position_table,
           token_type_table, W_proj, b_proj, tt_vis_table, pos_vis_table,
           ln_gamma, ln_beta):
    del image_loc
    tok_t = token_ids.T.astype(jnp.int32)
    gamma_row = ln_gamma.reshape(1, _H)
    beta_row = ln_beta.reshape(1, _H)

    fused_base = position_table[:_S] + token_type_table[0][None]
    vis_bias = (b_proj + tt_vis_table[1] + pos_vis_table[0]).reshape(1, _H)
    spos = (jnp.arange(_B, dtype=jnp.int32)[None, :] * _S
            + jnp.arange(_S, dtype=jnp.int32)[:, None])

    t_flat = _text_sc(word_table, tok_t, fused_base, spos, ln_gamma, ln_beta)

    v_out = _visual_tc(image_feat, W_proj, vis_bias, gamma_row, beta_row)

    return (t_flat.reshape(_B, _S, _H), v_out)


# speedup vs baseline: 1.0714x; 1.0714x over previous
import functools

import jax
import jax.numpy as jnp
from jax import lax
from jax.experimental import pallas as pl
from jax.experimental.pallas import tpu as pltpu
from jax.experimental.pallas import tpu_sc as plsc

_EPS = 1e-12

_H = 1024
_B = 64
_S = 128
_NB = 196
_VFEAT = 2048

_NTOK = _B * _S
_NW = 32
_SPW = _S // _NW
_NL = _H // 16


def _rsqrt_nr(v):
    i = lax.bitcast_convert_type(v, jnp.int32)
    y = lax.bitcast_convert_type(jnp.int32(0x5F3759DF) - (i >> 1), jnp.float32)
    half = v * 0.5
    for _ in range(3):
        y = y * (1.5 - half * y * y)
    return y


_HC = 32
_NHC = 8


def _text_sc(word_table, tok3, fused_base, spos3, gamma, beta):
    mesh = plsc.VectorSubcoreMesh(core_axis_name="c", subcore_axis_name="s")

    @functools.partial(
        pl.kernel,
        mesh=mesh,
        out_type=jax.ShapeDtypeStruct((_NTOK, _H), jnp.float32),
        scratch_types=[
            pltpu.VMEM((_NHC, _HC), jnp.int32),
            pltpu.VMEM((_NHC, _HC), jnp.int32),
            pltpu.VMEM((_HC, _H), jnp.float32),
            pltpu.VMEM((_HC, _H), jnp.float32),
            pltpu.VMEM((_SPW, _H), jnp.float32),
            pltpu.VMEM((_H,), jnp.float32),
            pltpu.VMEM((_H,), jnp.float32),
            pltpu.SemaphoreType.DMA,
            pltpu.SemaphoreType.DMA,
        ],
    )
    def k(table_hbm, tok_hbm, fb_hbm, spos_hbm, g_hbm, bt_hbm, out_hbm,
          idx_v, spos_v, buf_a, buf_b, base_v, gam_v, bet_v, gsem, ssem):
        wid = lax.axis_index("s") * 2 + lax.axis_index("c")
        pltpu.sync_copy(tok_hbm.at[wid], idx_v)
        pltpu.sync_copy(spos_hbm.at[wid], spos_v)
        pltpu.sync_copy(fb_hbm.at[pl.ds(wid * _SPW, _SPW)], base_v)
        pltpu.sync_copy(g_hbm, gam_v)
        pltpu.sync_copy(bt_hbm, bet_v)

        inv_h = jnp.full((16,), 1.0 / _H, jnp.float32)
        eps_v = jnp.full((16,), _EPS, jnp.float32)
        _dnums = lax.GatherDimensionNumbers(
            offset_dims=(), collapsed_slice_dims=(0,), start_index_map=(0,))
        _lane = lax.iota(jnp.int32, 16)
        _xor_idx = [(_lane ^ kk).reshape(16, 1) for kk in (8, 4, 2, 1)]

        def _splat_sum(v):
            for xi in _xor_idx:
                v = v + lax.gather(
                    v, xi, _dnums, (1,),
                    mode=lax.GatherScatterMode.PROMISE_IN_BOUNDS)
            return v

        bufs = (buf_a, buf_b)
        gathers = [None, None]
        scatters = [None, None]
        gathers[0] = pltpu.async_copy(table_hbm.at[idx_v.at[0]], buf_a, gsem)

        for h in range(_NHC):
            cur = bufs[h % 2]
            gathers[h % 2].wait()
            if h + 1 < _NHC:
                nb = (h + 1) % 2
                if scatters[nb] is not None:
                    scatters[nb].wait()
                    scatters[nb] = None
                gathers[nb] = pltpu.async_copy(
                    table_hbm.at[idx_v.at[h + 1]], bufs[nb], gsem)
            sl_idx = h // 2

            def row_body(r, _, cur=cur, sl_idx=sl_idx):
                a0 = jnp.zeros((16,), jnp.float32)
                a1 = jnp.zeros((16,), jnp.float32)
                a2 = jnp.zeros((16,), jnp.float32)
                a3 = jnp.zeros((16,), jnp.float32)
                b0 = jnp.zeros((16,), jnp.float32)
                b1 = jnp.zeros((16,), jnp.float32)
                b2 = jnp.zeros((16,), jnp.float32)
                b3 = jnp.zeros((16,), jnp.float32)
                accs = [a0, a1, a2, a3]
                acc2s = [b0, b1, b2, b3]
                for q in range(_NL):
                    sl = pl.ds(q * 16, 16)
                    x = cur[r, sl] + base_v[sl_idx, sl]
                    cur[r, sl] = x
                    accs[q % 4] = accs[q % 4] + x
                    acc2s[q % 4] = acc2s[q % 4] + x * x
                acc = (accs[0] + accs[1]) + (accs[2] + accs[3])
                acc2 = (acc2s[0] + acc2s[1]) + (acc2s[2] + acc2s[3])
                mu = _splat_sum(acc) * inv_h
                mu2 = _splat_sum(acc2) * inv_h
                rinv = _rsqrt_nr(mu2 - mu * mu + eps_v)
                for q in range(_NL):
                    sl = pl.ds(q * 16, 16)
                    cur[r, sl] = ((cur[r, sl] - mu) * rinv * gam_v[sl]
                                  + bet_v[sl])
                return _

            lax.fori_loop(0, _HC, row_body, None)
            scatters[h % 2] = pltpu.async_copy(
                cur, out_hbm.at[spos_v.at[h]], ssem)

        scatters[0].wait()
        scatters[1].wait()

    return k(word_table, tok3, fused_base, spos3, gamma, beta)


def _visual_tc(x3, w, bias_row, gamma_row, beta_row):
    k, n = _VFEAT, _H
    bb = 4
    grid = (_B // bb,)

    def body(x_ref, w_ref, b_ref, g_ref, bt_ref, o_ref):
        for j in range(bb):
            acc = jnp.dot(x_ref[j], w_ref[...],
                          preferred_element_type=jnp.float32)
            y = acc + b_ref[...]
            mu = jnp.mean(y, axis=-1, keepdims=True)
            var = jnp.mean((y - mu) ** 2, axis=-1, keepdims=True)
            o_ref[j] = (y - mu) * lax.rsqrt(var + _EPS) * g_ref[...] + bt_ref[...]

    return pl.pallas_call(
        body,
        grid=grid,
        in_specs=[
            pl.BlockSpec((bb, _NB, k), lambda i: (i, 0, 0)),
            pl.BlockSpec((k, n), lambda i: (0, 0)),
            pl.BlockSpec((1, n), lambda i: (0, 0)),
            pl.BlockSpec((1, n), lambda i: (0, 0)),
            pl.BlockSpec((1, n), lambda i: (0, 0)),
        ],
        out_specs=pl.BlockSpec((bb, _NB, n), lambda i: (i, 0, 0)),
        out_shape=jax.ShapeDtypeStruct((_B, _NB, n), jnp.float32),
    )(x3, w, bias_row, gamma_row, beta_row)


def kernel(token_ids, image_feat, image_loc, word_table, position_table,
           token_type_table, W_proj, b_proj, tt_vis_table, pos_vis_table,
           ln_gamma, ln_beta):
    del image_loc
    tok3 = (token_ids.T.astype(jnp.int32)
            .reshape(_NW, _SPW * 2, _HC))
    gamma_row = ln_gamma.reshape(1, _H)
    beta_row = ln_beta.reshape(1, _H)

    fused_base = position_table[:_S] + token_type_table[0][None]
    vis_bias = (b_proj + tt_vis_table[1] + pos_vis_table[0]).reshape(1, _H)
    spos = (jnp.arange(_B, dtype=jnp.int32)[None, :] * _S
            + jnp.arange(_S, dtype=jnp.int32)[:, None])
    spos3 = spos.reshape(_NW, _SPW * 2, _HC)

    t_flat = _text_sc(word_table, tok3, fused_base, spos3, ln_gamma, ln_beta)

    v_out = _visual_tc(image_feat, W_proj, vis_bias, gamma_row, beta_row)

    return (t_flat.reshape(_B, _S, _H), v_out)
